# baseline (speedup 1.0000x reference)
import functools

import jax
import jax.numpy as jnp
from jax import lax
from jax.experimental import pallas as pl
from jax.experimental.pallas import tpu as pltpu
from jax.experimental.pallas import tpu_sc as plsc

NC = 2
NS = 16
NW = NC * NS
CHUNK = 128


@functools.cache
def _make_gather(n_chunks, d, dtype):
  cpw = n_chunks // NW
  mesh = plsc.VectorSubcoreMesh(core_axis_name="c", subcore_axis_name="s")

  def body(idx_hbm, table_hbm, out_hbm, idx_v, rows_v, sem):
    wid = lax.axis_index("s") * NC + lax.axis_index("c")
    base = wid * cpw
    pltpu.sync_copy(idx_hbm.at[pl.ds(base, cpw)], idx_v)
    copies = [
        pltpu.async_copy(table_hbm.at[idx_v.at[j]], rows_v.at[j], sem)
        for j in range(cpw)
    ]
    for c in copies:
      c.wait()
    pltpu.sync_copy(rows_v, out_hbm.at[pl.ds(base, cpw)])

  return pl.kernel(
      body,
      out_type=jax.ShapeDtypeStruct((n_chunks, CHUNK, d), dtype),
      mesh=mesh,
      scratch_types=[
          pltpu.VMEM((cpw, CHUNK), jnp.int32),
          pltpu.VMEM((cpw, CHUNK, d), dtype),
          pltpu.SemaphoreType.DMA,
      ],
      compiler_params=pltpu.CompilerParams(use_tc_tiling_on_sc=False),
  )


def kernel(user_indices, embedding_table):
  (b,) = user_indices.shape
  _, d = embedding_table.shape
  n_chunks = b // CHUNK
  idx = user_indices.astype(jnp.int32).reshape(n_chunks, CHUNK)
  out = _make_gather(n_chunks, d, embedding_table.dtype)(idx, embedding_table)
  return out.reshape(b, d)


# speedup vs baseline: 1.7287x; 1.7287x over previous
import functools

import jax
import jax.numpy as jnp
from jax import lax
from jax.experimental import pallas as pl
from jax.experimental.pallas import tpu as pltpu
from jax.experimental.pallas import tpu_sc as plsc

NC = 2
NS = 16
NW = NC * NS


@functools.cache
def _make_gather(v, d, n):
  cpw = n // NW
  mesh = plsc.VectorSubcoreMesh(core_axis_name="c", subcore_axis_name="s")

  def body(idx_hbm, tab_hbm, out_hbm, idxv, ostage, gsem):
    wid = lax.axis_index("s") * NC + lax.axis_index("c")
    base = wid * cpw

    pltpu.sync_copy(idx_hbm.at[pl.ds(base, cpw)], idxv)

    def fire(ch, carry):
      vec = idxv[pl.ds(ch * 16, 16)]
      for l in range(16):
        pltpu.async_copy(
            tab_hbm.at[vec[l]], ostage.at[ch * 2 + l // 8, l % 8], gsem)
      return carry

    lax.fori_loop(0, cpw // 16, fire, 0)

    def drain(ch, carry):
      vec = idxv[pl.ds(ch * 16, 16)]
      for l in range(16):
        pltpu.make_async_copy(
            tab_hbm.at[vec[l]], ostage.at[ch * 2 + l // 8, l % 8], gsem).wait()
      return carry

    lax.fori_loop(0, cpw // 16, drain, 0)
    pltpu.sync_copy(ostage, out_hbm.at[pl.ds(base // 8, cpw // 8)])

  return pl.kernel(
      body,
      out_type=jax.ShapeDtypeStruct((n // 8, 8, d), jnp.float32),
      mesh=mesh,
      scratch_types=[
          pltpu.VMEM((cpw,), jnp.int32),
          pltpu.VMEM((cpw // 8, 8, d), jnp.float32),
          pltpu.SemaphoreType.DMA,
      ],
  )


def kernel(user_indices, embedding_table):
  (n,) = user_indices.shape
  v, d = embedding_table.shape
  idx = user_indices.astype(jnp.int32)
  out3 = _make_gather(v, d, n)(idx, embedding_table)
  return out3.reshape(n, d)
